# Initial kernel scaffold; baseline (speedup 1.0000x reference)
import math

import jax
import jax.numpy as jnp
from jax import lax
from jax.experimental import pallas as pl
from jax.experimental.pallas import tpu as pltpu
from jax.experimental.pallas import tpu_sc as plsc

N = 10000
E = 320000
M = 20000
G = 512
D = 128
R = 8

NC, NS = 2, 16
K = 128
NCHUNK = -(-E // (NC * NS * K))
E_PAD = NC * NS * NCHUNK * K
N_PAD = 10016
ROWS_PT = N_PAD // NS
W = 144

MCH = 2000
NCH = M // MCH

CS = 1250
CH = N // CS


def _a1_body(x_ref, wg_ref, ws_ref, rel_ref, tab_ref, xs_ref):
    r = pl.program_id(0)
    xg = jnp.dot(x_ref[...], wg_ref[...], preferred_element_type=jnp.float32)
    s = rel_ref[r]
    tab_ref[0, :, 0:D] = xg * s
    row = lax.broadcasted_iota(jnp.int32, (N_PAD, W - D), 0)
    lane = lax.broadcasted_iota(jnp.int32, (N_PAD, W - D), 1)
    tab_ref[0, :, D:W] = jnp.where((lane == 0) & (row < N), 1.0, 0.0)

    @pl.when(r == 0)
    def _():
        xs_ref[...] = jnp.dot(x_ref[...], ws_ref[...],
                              preferred_element_type=jnp.float32)


def _a2_body(f_ref, fb_ref, wf_ref, kf_ref, fsum, cnt):
    i = pl.program_id(0)

    @pl.when(i == 0)
    def _():
        fsum[...] = jnp.zeros_like(fsum)
        cnt[...] = jnp.zeros_like(cnt)

    oh = (fb_ref[...] == lax.broadcasted_iota(jnp.int32, (1, G), 1))
    ohf = oh.astype(jnp.float32)
    fsum[...] += lax.dot_general(ohf, f_ref[...], (((0,), (0,)), ((), ())),
                                 preferred_element_type=jnp.float32)
    cnt[...] += lax.dot_general(ohf, jnp.ones((MCH, 1), jnp.float32),
                                (((0,), (0,)), ((), ())),
                                preferred_element_type=jnp.float32)

    @pl.when(i == NCH - 1)
    def _():
        fpool = fsum[...] / jnp.maximum(cnt[...], 1.0)
        kf_ref[...] = jnp.dot(fpool, wf_ref[...],
                              preferred_element_type=jnp.float32)


def _sc_body(tab_ref, gidx_ref, sidx_ref, zero_ref, out_ref,
             gidx_v, sidx_v, rows_v, agg_sp, sem):
    cid = lax.axis_index("c")
    sid = lax.axis_index("s")
    base = sid * ROWS_PT
    pltpu.sync_copy(zero_ref.at[pl.ds(base, ROWS_PT)],
                    agg_sp.at[pl.ds(base, ROWS_PT)])
    pltpu.sync_copy(gidx_ref.at[cid, sid], gidx_v)
    pltpu.sync_copy(sidx_ref.at[cid, sid], sidx_v)
    plsc.subcore_barrier()

    def chunk(j, carry):
        pltpu.async_copy(tab_ref.at[gidx_v.at[j]], rows_v, sem).wait()
        pltpu.sync_copy(rows_v, agg_sp.at[sidx_v.at[j]], add=True)
        return carry

    lax.fori_loop(0, NCHUNK, chunk, 0)
    plsc.subcore_barrier()
    pltpu.sync_copy(agg_sp.at[pl.ds(base, ROWS_PT)],
                    out_ref.at[cid, pl.ds(base, ROWS_PT)])


def _c_body(agg_ref, xs_ref, xb_ref, kf_ref, wh_ref, wl_ref,
            z_ref, al_ref, e_scr, ex_scr, emax_scr, den_scr):
    kf = kf_ref[...]
    wh = wh_ref[...]
    wl = wl_ref[...]
    iota_g = lax.broadcasted_iota(jnp.int32, (1, G), 1)
    inv_sqrt_d = 1.0 / math.sqrt(float(D))

    def onehot(i):
        return xb_ref[i * CS:(i + 1) * CS, :] == iota_g

    def make_h(i):
        a0 = agg_ref[0, i * CS:(i + 1) * CS, :]
        a1 = agg_ref[1, i * CS:(i + 1) * CS, :]
        agg = a0[:, 0:D] + a1[:, 0:D]
        deg = a0[:, D:D + 1] + a1[:, D:D + 1]
        return agg / jnp.maximum(deg, 1.0) + xs_ref[i * CS:(i + 1) * CS, :]

    for i in range(CH):
        p = onehot(i)
        h = make_h(i)
        q = jnp.dot(h, wh, preferred_element_type=jnp.float32)
        kfx = jnp.dot(p.astype(jnp.float32), kf,
                      preferred_element_type=jnp.float32)
        e = jnp.sum(q * kfx, axis=1, keepdims=True) * inv_sqrt_d
        e_scr[:, i:i + 1] = e
        m = jnp.max(jnp.where(p, e, -1e30), axis=0, keepdims=True)
        if i == 0:
            emax_scr[...] = m
        else:
            emax_scr[...] = jnp.maximum(emax_scr[...], m)

    for i in range(CH):
        p = onehot(i)
        em = jnp.max(jnp.where(p, emax_scr[...], -1e30), axis=1, keepdims=True)
        ex = jnp.exp(e_scr[:, i:i + 1] - em)
        ex_scr[:, i:i + 1] = ex
        dp = jnp.sum(jnp.where(p, ex, 0.0), axis=0, keepdims=True)
        if i == 0:
            den_scr[...] = dp
        else:
            den_scr[...] += dp

    for i in range(CH):
        p = onehot(i)
        den = jnp.sum(jnp.where(p, den_scr[...], 0.0), axis=1, keepdims=True)
        alpha = ex_scr[:, i:i + 1] / den
        kfx = jnp.dot(p.astype(jnp.float32), kf,
                      preferred_element_type=jnp.float32)
        h = make_h(i)
        z_ref[i * CS:(i + 1) * CS, :] = jnp.dot(
            h + alpha * kfx, wl, preferred_element_type=jnp.float32)
        al_ref[i * CS:(i + 1) * CS, :] = alpha


def kernel(f, x, edge_index, edge_type, f_batch, x_batch,
           W_gcn, W_self, rel_scale, Wf, Wh, W_lin):
    x_pad = jnp.concatenate([x, jnp.zeros((N_PAD - N, D), jnp.float32)], axis=0)
    gidx = edge_type * N_PAD + edge_index[0]
    sidx = edge_index[1]
    pad = E_PAD - E
    gidx = jnp.concatenate([gidx, jnp.full((pad,), N, jnp.int32)])
    sidx = jnp.concatenate([sidx, jnp.full((pad,), N, jnp.int32)])
    gidx = gidx.reshape(NC, NS, NCHUNK, K)
    sidx = sidx.reshape(NC, NS, NCHUNK, K)
    zeros = jnp.zeros((N_PAD, W), jnp.float32)
    fb2 = f_batch.reshape(M, 1)
    xb2 = x_batch.reshape(N, 1)

    tab, xs = pl.pallas_call(
        _a1_body,
        grid=(R,),
        in_specs=[
            pl.BlockSpec((N_PAD, D), lambda r: (0, 0)),
            pl.BlockSpec((D, D), lambda r: (0, 0)),
            pl.BlockSpec((D, D), lambda r: (0, 0)),
            pl.BlockSpec(memory_space=pltpu.SMEM),
        ],
        out_specs=[
            pl.BlockSpec((1, N_PAD, W), lambda r: (r, 0, 0)),
            pl.BlockSpec((N_PAD, D), lambda r: (0, 0)),
        ],
        out_shape=[
            jax.ShapeDtypeStruct((R, N_PAD, W), jnp.float32),
            jax.ShapeDtypeStruct((N_PAD, D), jnp.float32),
        ],
    )(x_pad, W_gcn, W_self, rel_scale)

    kf = pl.pallas_call(
        _a2_body,
        grid=(NCH,),
        in_specs=[
            pl.BlockSpec((MCH, D), lambda i: (i, 0)),
            pl.BlockSpec((MCH, 1), lambda i: (i, 0)),
            pl.BlockSpec((D, D), lambda i: (0, 0)),
        ],
        out_specs=pl.BlockSpec((G, D), lambda i: (0, 0)),
        out_shape=jax.ShapeDtypeStruct((G, D), jnp.float32),
        scratch_shapes=[
            pltpu.VMEM((G, D), jnp.float32),
            pltpu.VMEM((G, 1), jnp.float32),
        ],
    )(f, fb2, Wf)

    mesh = plsc.VectorSubcoreMesh(core_axis_name="c", subcore_axis_name="s")
    agg2 = pl.kernel(
        _sc_body,
        out_type=jax.ShapeDtypeStruct((NC, N_PAD, W), jnp.float32),
        mesh=mesh,
        scratch_types=[
            pltpu.VMEM((NCHUNK, K), jnp.int32),
            pltpu.VMEM((NCHUNK, K), jnp.int32),
            pltpu.VMEM((K, W), jnp.float32),
            pltpu.VMEM_SHARED((N_PAD, W), jnp.float32),
            pltpu.SemaphoreType.DMA,
        ],
    )(tab, gidx, sidx, zeros)

    z, al = pl.pallas_call(
        _c_body,
        in_specs=[
            pl.BlockSpec((NC, N_PAD, W), lambda: (0, 0, 0)),
            pl.BlockSpec((N_PAD, D), lambda: (0, 0)),
            pl.BlockSpec((N, 1), lambda: (0, 0)),
            pl.BlockSpec((G, D), lambda: (0, 0)),
            pl.BlockSpec((D, D), lambda: (0, 0)),
            pl.BlockSpec((D, D), lambda: (0, 0)),
        ],
        out_specs=[
            pl.BlockSpec((N, D), lambda: (0, 0)),
            pl.BlockSpec((N, 1), lambda: (0, 0)),
        ],
        out_shape=[
            jax.ShapeDtypeStruct((N, D), jnp.float32),
            jax.ShapeDtypeStruct((N, 1), jnp.float32),
        ],
        scratch_shapes=[
            pltpu.VMEM((CS, CH), jnp.float32),
            pltpu.VMEM((CS, CH), jnp.float32),
            pltpu.VMEM((1, G), jnp.float32),
            pltpu.VMEM((1, G), jnp.float32),
        ],
    )(agg2, xs, xb2, kf, Wh, W_lin)

    return z, al.reshape(N)


# speedup vs baseline: 3.7805x; 3.7805x over previous
import math

import jax
import jax.numpy as jnp
from jax import lax
from jax.experimental import pallas as pl
from jax.experimental.pallas import tpu as pltpu
from jax.experimental.pallas import tpu_sc as plsc

N = 10000
E = 320000
M = 20000
G = 512
D = 128
R = 8

NC, NS = 2, 16
K = 128
NCHUNK = -(-E // (NC * NS * K))
E_PAD = NC * NS * NCHUNK * K
N_PAD = 10016
ROWS_PT = N_PAD // NS
W = 144

MCH = 2000
NCH = M // MCH

CS = 1250
CH = N // CS


def _a1_body(x_ref, wg_ref, ws_ref, rel_ref, tab_ref, xs_ref):
    r = pl.program_id(0)
    xg = jnp.dot(x_ref[...], wg_ref[...], preferred_element_type=jnp.float32)
    s = rel_ref[r]
    tab_ref[0, :, 0:D] = xg * s
    row = lax.broadcasted_iota(jnp.int32, (N_PAD, W - D), 0)
    lane = lax.broadcasted_iota(jnp.int32, (N_PAD, W - D), 1)
    tab_ref[0, :, D:W] = jnp.where((lane == 0) & (row < N), 1.0, 0.0)

    @pl.when(r == 0)
    def _():
        xs_ref[...] = jnp.dot(x_ref[...], ws_ref[...],
                              preferred_element_type=jnp.float32)


def _a2_body(f_ref, fb_ref, wf_ref, kf_ref, fsum, cnt):
    i = pl.program_id(0)

    @pl.when(i == 0)
    def _():
        fsum[...] = jnp.zeros_like(fsum)
        cnt[...] = jnp.zeros_like(cnt)

    oh = (fb_ref[...] == lax.broadcasted_iota(jnp.int32, (1, G), 1))
    ohf = oh.astype(jnp.float32)
    fsum[...] += lax.dot_general(ohf, f_ref[...], (((0,), (0,)), ((), ())),
                                 preferred_element_type=jnp.float32)
    cnt[...] += lax.dot_general(ohf, jnp.ones((MCH, 1), jnp.float32),
                                (((0,), (0,)), ((), ())),
                                preferred_element_type=jnp.float32)

    @pl.when(i == NCH - 1)
    def _():
        fpool = fsum[...] / jnp.maximum(cnt[...], 1.0)
        kf_ref[...] = jnp.dot(fpool, wf_ref[...],
                              preferred_element_type=jnp.float32)


def _sc_body(tab_ref, gidx_ref, sidx_ref, zero_ref, out_ref,
             gidx_v, sidx_v, rows_v, agg_sp, sem):
    cid = lax.axis_index("c")
    sid = lax.axis_index("s")
    base = sid * ROWS_PT
    pltpu.sync_copy(zero_ref.at[pl.ds(base, ROWS_PT)],
                    agg_sp.at[pl.ds(base, ROWS_PT)])
    pltpu.sync_copy(gidx_ref.at[cid, sid], gidx_v)
    pltpu.sync_copy(sidx_ref.at[cid, sid], sidx_v)
    plsc.subcore_barrier()

    def chunk(j, carry):
        pltpu.async_copy(tab_ref.at[gidx_v.at[j]], rows_v, sem).wait()
        pltpu.sync_copy(rows_v, agg_sp.at[sidx_v.at[j]], add=True)
        return carry

    lax.fori_loop(0, NCHUNK, chunk, 0)
    plsc.subcore_barrier()
    pltpu.sync_copy(agg_sp.at[pl.ds(base, ROWS_PT)],
                    out_ref.at[cid, pl.ds(base, ROWS_PT)])


def _c_body(agg_ref, xs_ref, xb_ref, kf_ref, wh_ref, wl_ref,
            z_ref, al_ref, e_scr, ex_scr, emax_scr, den_scr):
    kf = kf_ref[...]
    wh = wh_ref[...]
    wl = wl_ref[...]
    iota_g = lax.broadcasted_iota(jnp.int32, (1, G), 1)
    inv_sqrt_d = 1.0 / math.sqrt(float(D))

    def onehot(i):
        return xb_ref[i * CS:(i + 1) * CS, :] == iota_g

    def make_h(i):
        a0 = agg_ref[0, i * CS:(i + 1) * CS, :]
        a1 = agg_ref[1, i * CS:(i + 1) * CS, :]
        agg = a0[:, 0:D] + a1[:, 0:D]
        deg = a0[:, D:D + 1] + a1[:, D:D + 1]
        return agg / jnp.maximum(deg, 1.0) + xs_ref[i * CS:(i + 1) * CS, :]

    for i in range(CH):
        p = onehot(i)
        h = make_h(i)
        q = jnp.dot(h, wh, preferred_element_type=jnp.float32)
        kfx = jnp.dot(p.astype(jnp.float32), kf,
                      preferred_element_type=jnp.float32)
        e = jnp.sum(q * kfx, axis=1, keepdims=True) * inv_sqrt_d
        e_scr[:, i:i + 1] = e
        m = jnp.max(jnp.where(p, e, -1e30), axis=0, keepdims=True)
        if i == 0:
            emax_scr[...] = m
        else:
            emax_scr[...] = jnp.maximum(emax_scr[...], m)

    for i in range(CH):
        p = onehot(i)
        em = jnp.max(jnp.where(p, emax_scr[...], -1e30), axis=1, keepdims=True)
        ex = jnp.exp(e_scr[:, i:i + 1] - em)
        ex_scr[:, i:i + 1] = ex
        dp = jnp.sum(jnp.where(p, ex, 0.0), axis=0, keepdims=True)
        if i == 0:
            den_scr[...] = dp
        else:
            den_scr[...] += dp

    for i in range(CH):
        p = onehot(i)
        den = jnp.sum(jnp.where(p, den_scr[...], 0.0), axis=1, keepdims=True)
        alpha = ex_scr[:, i:i + 1] / den
        kfx = jnp.dot(p.astype(jnp.float32), kf,
                      preferred_element_type=jnp.float32)
        h = make_h(i)
        z_ref[i * CS:(i + 1) * CS, :] = jnp.dot(
            h + alpha * kfx, wl, preferred_element_type=jnp.float32)
        al_ref[i * CS:(i + 1) * CS, :] = alpha


def kernel(f, x, edge_index, edge_type, f_batch, x_batch,
           W_gcn, W_self, rel_scale, Wf, Wh, W_lin):
    x_pad = jnp.concatenate([x, jnp.zeros((N_PAD - N, D), jnp.float32)], axis=0)
    gidx = edge_type * N_PAD + edge_index[0]
    sidx = edge_index[1]
    pad = E_PAD - E
    gidx = jnp.concatenate([gidx, jnp.full((pad,), N, jnp.int32)])
    sidx = jnp.concatenate([sidx, jnp.full((pad,), N, jnp.int32)])
    gidx = gidx.reshape(NC, NS, NCHUNK, K)
    sidx = sidx.reshape(NC, NS, NCHUNK, K)
    zeros = jnp.zeros((N_PAD, W), jnp.float32)
    fb2 = f_batch.reshape(M, 1)
    xb2 = x_batch.reshape(N, 1)

    tab, xs = pl.pallas_call(
        _a1_body,
        grid=(R,),
        in_specs=[
            pl.BlockSpec((N_PAD, D), lambda r: (0, 0)),
            pl.BlockSpec((D, D), lambda r: (0, 0)),
            pl.BlockSpec((D, D), lambda r: (0, 0)),
            pl.BlockSpec(memory_space=pltpu.SMEM),
        ],
        out_specs=[
            pl.BlockSpec((1, N_PAD, W), lambda r: (r, 0, 0)),
            pl.BlockSpec((N_PAD, D), lambda r: (0, 0)),
        ],
        out_shape=[
            jax.ShapeDtypeStruct((R, N_PAD, W), jnp.float32),
            jax.ShapeDtypeStruct((N_PAD, D), jnp.float32),
        ],
    )(x_pad, W_gcn, W_self, rel_scale)

    kf = pl.pallas_call(
        _a2_body,
        grid=(NCH,),
        in_specs=[
            pl.BlockSpec((MCH, D), lambda i: (i, 0)),
            pl.BlockSpec((MCH, 1), lambda i: (i, 0)),
            pl.BlockSpec((D, D), lambda i: (0, 0)),
        ],
        out_specs=pl.BlockSpec((G, D), lambda i: (0, 0)),
        out_shape=jax.ShapeDtypeStruct((G, D), jnp.float32),
        scratch_shapes=[
            pltpu.VMEM((G, D), jnp.float32),
            pltpu.VMEM((G, 1), jnp.float32),
        ],
    )(f, fb2, Wf)

    mesh = plsc.VectorSubcoreMesh(core_axis_name="c", subcore_axis_name="s")
    agg2 = pl.kernel(
        _sc_body,
        out_type=jax.ShapeDtypeStruct((NC, N_PAD, W), jnp.float32),
        mesh=mesh,
        compiler_params=pltpu.CompilerParams(use_tc_tiling_on_sc=False),
        scratch_types=[
            pltpu.VMEM((NCHUNK, K), jnp.int32),
            pltpu.VMEM((NCHUNK, K), jnp.int32),
            pltpu.VMEM((K, W), jnp.float32),
            pltpu.VMEM_SHARED((N_PAD, W), jnp.float32),
            pltpu.SemaphoreType.DMA,
        ],
    )(tab.reshape(R * N_PAD, W), gidx, sidx, zeros)

    z, al = pl.pallas_call(
        _c_body,
        in_specs=[
            pl.BlockSpec((NC, N_PAD, W), lambda: (0, 0, 0)),
            pl.BlockSpec((N_PAD, D), lambda: (0, 0)),
            pl.BlockSpec((N, 1), lambda: (0, 0)),
            pl.BlockSpec((G, D), lambda: (0, 0)),
            pl.BlockSpec((D, D), lambda: (0, 0)),
            pl.BlockSpec((D, D), lambda: (0, 0)),
        ],
        out_specs=[
            pl.BlockSpec((N, D), lambda: (0, 0)),
            pl.BlockSpec((N, 1), lambda: (0, 0)),
        ],
        out_shape=[
            jax.ShapeDtypeStruct((N, D), jnp.float32),
            jax.ShapeDtypeStruct((N, 1), jnp.float32),
        ],
        scratch_shapes=[
            pltpu.VMEM((CS, CH), jnp.float32),
            pltpu.VMEM((CS, CH), jnp.float32),
            pltpu.VMEM((1, G), jnp.float32),
            pltpu.VMEM((1, G), jnp.float32),
        ],
    )(agg2, xs, xb2, kf, Wh, W_lin)

    return z, al.reshape(N)
